# baseline (speedup 1.0000x reference)
import functools

import jax
import jax.numpy as jnp
from jax import lax
from jax.experimental import pallas as pl
from jax.experimental.pallas import tpu as pltpu
from jax.experimental.pallas import tpu_sc as plsc

N = 10000
NP = 10240
E = 320000
D = 128

NC = 2
NS = 16
NW = NC * NS
B = 88
NB = 114
NPAIR = NB // 2
EPW = NB * B
EP = NW * EPW
RPT = NP // NS
DEGW = 16
DUMMY_DST = NP - 1


DZR = 40
ZCH = 64


def _sc_agg_body(compute_deg, h_hbm, edge_hbm, agg_out, deg_out,
                 agg_sh, deg_sh, sp0, dp0, sp1, dp1, sp2, dp2,
                 rows0, rows1, rows2, ones_v, dzbuf,
                 psem0, psem1, psem2, gsem0, gsem1, gsem2,
                 ssem0, ssem1, ssem2):
    cid = lax.axis_index("c")
    sid = lax.axis_index("s")
    wid = cid * NS + sid

    def _zb(i, _):
        r = i // (D // 16)
        c = i % (D // 16)
        rows0[r, pl.ds(c * 16, 16)] = jnp.zeros((16,), jnp.float32)
        return 0
    lax.fori_loop(0, B * (D // 16), _zb, 0)
    for j in range(RPT // ZCH):
        pltpu.sync_copy(rows0.at[pl.ds(0, ZCH)],
                        agg_sh.at[pl.ds(sid * RPT + j * ZCH, ZCH)])

    if compute_deg:
        def _ob(i, _):
            ones_v[i] = jnp.ones((16,), jnp.float32)
            return 0
        lax.fori_loop(0, B, _ob, 0)

        def _dz(i, _):
            dzbuf[i] = jnp.zeros((16,), jnp.float32)
            return 0
        lax.fori_loop(0, DZR, _dz, 0)
        for j in range(RPT // DZR):
            pltpu.sync_copy(dzbuf, deg_sh.at[pl.ds(sid * RPT + j * DZR, DZR)])

    plsc.subcore_barrier()

    prs = ((sp0, dp0, psem0), (sp1, dp1, psem1), (sp2, dp2, psem2))
    rws = ((rows0, gsem0, ssem0), (rows1, gsem1, ssem1), (rows2, gsem2, ssem2))

    def _pair_start(p, pr):
        pltpu.async_copy(edge_hbm.at[0, wid, p], pr[0], pr[2])
        pltpu.async_copy(edge_hbm.at[1, wid, p], pr[1], pr[2])

    def _pair_wait(p, pr):
        pltpu.make_async_copy(edge_hbm.at[0, wid, p], pr[0], pr[2]).wait()
        pltpu.make_async_copy(edge_hbm.at[1, wid, p], pr[1], pr[2]).wait()

    def _scatter_start(rw, dv):
        pltpu.async_copy(rw[0], agg_sh.at[dv], rw[2], add=True)
        if compute_deg:
            pltpu.async_copy(ones_v, deg_sh.at[dv], rw[2], add=True)

    def _scatter_wait(rw, dv):
        pltpu.make_async_copy(rw[0], agg_sh.at[dv], rw[2]).wait()
        if compute_deg:
            pltpu.make_async_copy(ones_v, deg_sh.at[dv], rw[2]).wait()

    _pair_start(0, prs[0])
    _pair_start(1, prs[1])
    _pair_wait(0, prs[0])
    pltpu.async_copy(h_hbm.at[prs[0][0].at[0]], rows0, gsem0)

    def _step(k, half, k_mod, rw_cur, rw_nxt, rw_old, pr_cur, pr_nxt, pr_old):
        kk = k + 1
        @pl.when(kk < NB)
        def _():
            if half == 1:
                _pair_wait(kk // 2, pr_nxt)
            pltpu.async_copy(h_hbm.at[pr_nxt[0].at[1 - half]], rw_nxt[0],
                             rw_nxt[1])
        pltpu.make_async_copy(h_hbm.at[pr_cur[0].at[half]], rw_cur[0],
                              rw_cur[1]).wait()
        _scatter_start(rw_cur, pr_cur[1].at[half])
        @pl.when(k >= 1)
        def _():
            _scatter_wait(rw_old, pr_old[1].at[1 - half])
        if half == 0:
            pbuf = prs[(k_mod // 2 + 2) % 3]
            p = k // 2 + 2
            @pl.when(2 * p < NB)
            def _():
                _pair_start(p, pbuf)

    def _six(g, _):
        for m in range(6):
            k = 6 * g + m
            _step(k, m % 2, m,
                  rws[m % 3], rws[(m + 1) % 3], rws[(m + 2) % 3],
                  prs[(m // 2) % 3], prs[((m + 1) // 2) % 3],
                  prs[((m - 1) // 2) % 3])
        return 0
    lax.fori_loop(0, NB // 6, _six, 0)

    _scatter_wait(rws[(NB - 1) % 3], prs[((NB - 1) // 2) % 3][1].at[1])

    plsc.subcore_barrier()

    pltpu.sync_copy(agg_sh.at[pl.ds(sid * RPT, RPT)],
                    agg_out.at[cid, pl.ds(sid * RPT, RPT)])
    if compute_deg:
        pltpu.sync_copy(deg_sh.at[pl.ds(sid * RPT, RPT)],
                        deg_out.at[cid, pl.ds(sid * RPT, RPT)])


def _make_sc_agg(compute_deg):
    out_type = [jax.ShapeDtypeStruct((NC, NP, D), jnp.float32)]
    if compute_deg:
        out_type.append(jax.ShapeDtypeStruct((NC, NP, DEGW), jnp.float32))
    scratch = [
        pltpu.VMEM_SHARED((NP, D), jnp.float32),
        pltpu.VMEM_SHARED((NP, DEGW), jnp.float32),
        pltpu.VMEM((2, B), jnp.int32),
        pltpu.VMEM((2, B), jnp.int32),
        pltpu.VMEM((2, B), jnp.int32),
        pltpu.VMEM((2, B), jnp.int32),
        pltpu.VMEM((2, B), jnp.int32),
        pltpu.VMEM((2, B), jnp.int32),
        pltpu.VMEM((B, D), jnp.float32),
        pltpu.VMEM((B, D), jnp.float32),
        pltpu.VMEM((B, D), jnp.float32),
        pltpu.VMEM((B, DEGW), jnp.float32),
        pltpu.VMEM((DZR, DEGW), jnp.float32),
        pltpu.SemaphoreType.DMA,
        pltpu.SemaphoreType.DMA,
        pltpu.SemaphoreType.DMA,
        pltpu.SemaphoreType.DMA,
        pltpu.SemaphoreType.DMA,
        pltpu.SemaphoreType.DMA,
        pltpu.SemaphoreType.DMA,
        pltpu.SemaphoreType.DMA,
        pltpu.SemaphoreType.DMA,
    ]
    mesh = plsc.VectorSubcoreMesh(core_axis_name="c", subcore_axis_name="s")
    if compute_deg:
        def body(h, e, agg_o, deg_o, *s):
            _sc_agg_body(True, h, e, agg_o, deg_o, *s)
    else:
        def body(h, e, agg_o, *s):
            _sc_agg_body(False, h, e, agg_o, None, *s)
    return pl.kernel(body, out_type=out_type, mesh=mesh,
                     scratch_types=scratch,
                     compiler_params=pltpu.CompilerParams(
                         use_tc_tiling_on_sc=False))


_sc_agg_deg = _make_sc_agg(True)
_sc_agg = _make_sc_agg(False)


R = 1024


RB = 1024 // 8


def _norm_agg(aggp, degp):
    agg = aggp[0] + aggp[1]
    d = degp[0] + degp[1]
    cc = lax.broadcasted_iota(jnp.int32, (128, 8), 0)
    kk = lax.broadcasted_iota(jnp.int32, (128, 8), 1)
    sel = (cc == kk * DEGW).astype(jnp.float32)
    dv = jnp.dot(d, sel, preferred_element_type=jnp.float32)
    dinv = 1.0 / jnp.maximum(dv, 1.0)
    a3 = agg.reshape(RB, 8, 128) * dinv[:, :, None]
    return a3.reshape(RB * 8, 128)


def _tc1_body(aggp, degp, w, b, out):
    h = jnp.dot(_norm_agg(aggp, degp), w[...],
                preferred_element_type=jnp.float32)
    out[...] = jnp.maximum(h + b[...], 0.0)


def _tc2_body(aggp, degp, w, b, lw1, lb1, lw2, lb2, out):
    h = jnp.dot(_norm_agg(aggp, degp), w[...],
                preferred_element_type=jnp.float32)
    h = jnp.maximum(h + b[...], 0.0)
    t = jnp.dot(h, lw1[...], preferred_element_type=jnp.float32) + lb1[...]
    out[...] = jnp.dot(t, lw2[...], preferred_element_type=jnp.float32) + lb2[...]


def _full(shape):
    return pl.BlockSpec(shape, lambda i: (0,) * len(shape))


_tc1 = pl.pallas_call(
    _tc1_body,
    grid=(NP // R,),
    in_specs=[
        pl.BlockSpec((NC, R, D), lambda i: (0, i, 0)),
        pl.BlockSpec((NC, RB, 128), lambda i: (0, i, 0)),
        _full((D, D)),
        _full((1, D)),
    ],
    out_specs=pl.BlockSpec((R, D), lambda i: (i, 0)),
    out_shape=jax.ShapeDtypeStruct((NP, D), jnp.float32),
)

_tc2 = pl.pallas_call(
    _tc2_body,
    grid=(NP // R,),
    in_specs=[
        pl.BlockSpec((NC, R, D), lambda i: (0, i, 0)),
        pl.BlockSpec((NC, RB, 128), lambda i: (0, i, 0)),
        _full((D, D)),
        _full((1, D)),
        _full((D, 256)),
        _full((1, 256)),
        _full((256, 64)),
        _full((1, 64)),
    ],
    out_specs=pl.BlockSpec((R, 64), lambda i: (i, 0)),
    out_shape=jax.ShapeDtypeStruct((N, 64), jnp.float32),
)


@jax.jit
def kernel(x, edge_index, W1, b1, W2, b2, LW1, Lb1, LW2, Lb2):
    pad_src = jnp.zeros((EP - E,), jnp.int32)
    pad_dst = jnp.full((EP - E,), DUMMY_DST, jnp.int32)
    edge_flat = jnp.concatenate(
        [edge_index[0], pad_src, edge_index[1], pad_dst]
    ).reshape(2, NW, NPAIR, 2, B)
    aggp1, degp = _sc_agg_deg(x, edge_flat)
    degv = degp.reshape(NC, NP * DEGW // 128, 128)
    h1 = _tc1(aggp1, degv, W1, b1.reshape(1, D))
    aggp2, = _sc_agg(h1, edge_flat)
    out = _tc2(aggp2, degv, W2, b2.reshape(1, D), LW1, Lb1.reshape(1, 256),
               LW2, Lb2.reshape(1, 64))
    return out


# speedup vs baseline: 1.1654x; 1.1654x over previous
import functools

import jax
import jax.numpy as jnp
from jax import lax
from jax.experimental import pallas as pl
from jax.experimental.pallas import tpu as pltpu
from jax.experimental.pallas import tpu_sc as plsc

N = 10000
NP = 10240
E = 320000
D = 128

NC = 2
NS = 16
NW = NC * NS
EPW = E // NW
B = 80
NB = EPW // B
RPT = NP // NS
DEGW = 8


def _sc_agg_body(compute_deg, args):
    if compute_deg:
        (h_hbm, edge_hbm, zagg_hbm, zdeg_hbm, ones_hbm, agg_out, deg_out,
         agg_sh, deg_sh,
         src0, dst0, src1, dst1, src2, dst2, src3, dst3,
         rows0, rows1, rows2, rows3, ones_v,
         isem0, isem1, isem2, isem3, gsem0, gsem1, gsem2, gsem3,
         ssem0, ssem1, ssem2, ssem3) = args
    else:
        (h_hbm, edge_hbm, zagg_hbm, agg_out,
         agg_sh, deg_sh,
         src0, dst0, src1, dst1, src2, dst2, src3, dst3,
         rows0, rows1, rows2, rows3, ones_v,
         isem0, isem1, isem2, isem3, gsem0, gsem1, gsem2, gsem3,
         ssem0, ssem1, ssem2, ssem3) = args
        deg_out = zdeg_hbm = ones_hbm = None
    cid = lax.axis_index("c")
    sid = lax.axis_index("s")
    wid = cid * NS + sid
    base = wid * EPW

    pltpu.sync_copy(zagg_hbm, agg_sh.at[pl.ds(sid * RPT, RPT)])
    if compute_deg:
        pltpu.sync_copy(zdeg_hbm, deg_sh.at[pl.ds(sid * RPT, RPT)])
        pltpu.sync_copy(ones_hbm, ones_v)

    plsc.subcore_barrier()

    slots = ((src0, dst0, rows0, isem0, gsem0, ssem0),
             (src1, dst1, rows1, isem1, gsem1, ssem1),
             (src2, dst2, rows2, isem2, gsem2, ssem2),
             (src3, dst3, rows3, isem3, gsem3, ssem3))

    def _idx_start(j, s):
        pltpu.async_copy(edge_hbm.at[pl.ds(base + j * B, B)], s[0], s[3])
        pltpu.async_copy(edge_hbm.at[pl.ds(E + base + j * B, B)], s[1], s[3])

    def _idx_wait(j, s):
        pltpu.make_async_copy(edge_hbm.at[pl.ds(base + j * B, B)], s[0],
                              s[3]).wait()
        pltpu.make_async_copy(edge_hbm.at[pl.ds(E + base + j * B, B)], s[1],
                              s[3]).wait()

    def _scatter_start(s):
        pltpu.async_copy(s[2], agg_sh.at[s[1]], s[5], add=True)
        if compute_deg:
            pltpu.async_copy(ones_v, deg_sh.at[s[1]], s[5], add=True)

    def _scatter_wait(s):
        pltpu.make_async_copy(s[2], agg_sh.at[s[1]], s[5]).wait()
        if compute_deg:
            pltpu.make_async_copy(ones_v, deg_sh.at[s[1]], s[5]).wait()

    _idx_start(0, slots[0])
    _idx_start(1, slots[1])
    _idx_wait(0, slots[0])
    pltpu.async_copy(h_hbm.at[slots[0][0]], slots[0][2], slots[0][4])

    def _step(k, s_cur, s_nxt, s_idx, s_old):
        @pl.when(k + 1 < NB)
        def _():
            _idx_wait(k + 1, s_nxt)
            pltpu.async_copy(h_hbm.at[s_nxt[0]], s_nxt[2], s_nxt[4])
        pltpu.make_async_copy(h_hbm.at[s_cur[0]], s_cur[2], s_cur[4]).wait()
        _scatter_start(s_cur)
        @pl.when(k >= 1)
        def _():
            _scatter_wait(s_old)
        @pl.when(k + 2 < NB)
        def _():
            _idx_start(k + 2, s_idx)

    def _quad(g, _):
        for m in range(4):
            k = 4 * g + m
            @pl.when(k < NB)
            def _():
                _step(k, slots[m], slots[(m + 1) % 4], slots[(m + 2) % 4],
                      slots[(m + 3) % 4])
        return 0
    lax.fori_loop(0, (NB + 3) // 4, _quad, 0)

    _scatter_wait(slots[(NB - 1) % 4])

    plsc.subcore_barrier()

    pltpu.sync_copy(agg_sh.at[pl.ds(sid * RPT, RPT)],
                    agg_out.at[cid, pl.ds(sid * RPT, RPT)])
    if compute_deg:
        pltpu.sync_copy(deg_sh.at[pl.ds(sid * RPT, RPT)],
                        deg_out.at[cid, pl.ds(sid * RPT, RPT)])


def _make_sc_agg(compute_deg):
    out_type = [jax.ShapeDtypeStruct((NC, NP, D), jnp.float32)]
    if compute_deg:
        out_type.append(jax.ShapeDtypeStruct((NC, NP, DEGW), jnp.float32))
    scratch = [
        pltpu.VMEM_SHARED((NP, D), jnp.float32),
        pltpu.VMEM_SHARED((NP, DEGW), jnp.float32),
        pltpu.VMEM((B,), jnp.int32),
        pltpu.VMEM((B,), jnp.int32),
        pltpu.VMEM((B,), jnp.int32),
        pltpu.VMEM((B,), jnp.int32),
        pltpu.VMEM((B,), jnp.int32),
        pltpu.VMEM((B,), jnp.int32),
        pltpu.VMEM((B,), jnp.int32),
        pltpu.VMEM((B,), jnp.int32),
        pltpu.VMEM((B, D), jnp.float32),
        pltpu.VMEM((B, D), jnp.float32),
        pltpu.VMEM((B, D), jnp.float32),
        pltpu.VMEM((B, D), jnp.float32),
        pltpu.VMEM((B, DEGW), jnp.float32),
        pltpu.SemaphoreType.DMA,
        pltpu.SemaphoreType.DMA,
        pltpu.SemaphoreType.DMA,
        pltpu.SemaphoreType.DMA,
        pltpu.SemaphoreType.DMA,
        pltpu.SemaphoreType.DMA,
        pltpu.SemaphoreType.DMA,
        pltpu.SemaphoreType.DMA,
        pltpu.SemaphoreType.DMA,
        pltpu.SemaphoreType.DMA,
        pltpu.SemaphoreType.DMA,
        pltpu.SemaphoreType.DMA,
    ]
    mesh = plsc.VectorSubcoreMesh(core_axis_name="c", subcore_axis_name="s")
    def body(*a):
        _sc_agg_body(compute_deg, a)
    return pl.kernel(body, out_type=out_type, mesh=mesh,
                     scratch_types=scratch,
                     compiler_params=pltpu.CompilerParams(
                         use_tc_tiling_on_sc=False))


_sc_agg_deg = _make_sc_agg(True)
_sc_agg = _make_sc_agg(False)


R = 1024


NPR = 128 // DEGW
RB = 1024 // NPR


def _norm_agg(aggp, degp):
    agg = aggp[0] + aggp[1]
    d = degp[0] + degp[1]
    cc = lax.broadcasted_iota(jnp.int32, (128, NPR), 0)
    kk = lax.broadcasted_iota(jnp.int32, (128, NPR), 1)
    sel = (cc == kk * DEGW).astype(jnp.float32)
    dv = jnp.dot(d, sel, preferred_element_type=jnp.float32)
    dinv = 1.0 / jnp.maximum(dv, 1.0)
    a3 = agg.reshape(RB, NPR, 128) * dinv[:, :, None]
    return a3.reshape(RB * NPR, 128)


def _tc1_body(aggp, degp, w, b, out):
    h = jnp.dot(_norm_agg(aggp, degp), w[...],
                preferred_element_type=jnp.float32)
    out[...] = jnp.maximum(h + b[...], 0.0)


def _tc2_body(aggp, degp, w, b, lw1, lb1, lw2, lb2, out):
    h = jnp.dot(_norm_agg(aggp, degp), w[...],
                preferred_element_type=jnp.float32)
    h = jnp.maximum(h + b[...], 0.0)
    t = jnp.dot(h, lw1[...], preferred_element_type=jnp.float32) + lb1[...]
    out[...] = jnp.dot(t, lw2[...], preferred_element_type=jnp.float32) + lb2[...]


def _full(shape):
    return pl.BlockSpec(shape, lambda i: (0,) * len(shape))


_tc1 = pl.pallas_call(
    _tc1_body,
    grid=(NP // R,),
    in_specs=[
        pl.BlockSpec((NC, R, D), lambda i: (0, i, 0)),
        pl.BlockSpec((NC, RB, 128), lambda i: (0, i, 0)),
        _full((D, D)),
        _full((1, D)),
    ],
    out_specs=pl.BlockSpec((R, D), lambda i: (i, 0)),
    out_shape=jax.ShapeDtypeStruct((NP, D), jnp.float32),
)

_tc2 = pl.pallas_call(
    _tc2_body,
    grid=(NP // R,),
    in_specs=[
        pl.BlockSpec((NC, R, D), lambda i: (0, i, 0)),
        pl.BlockSpec((NC, RB, 128), lambda i: (0, i, 0)),
        _full((D, D)),
        _full((1, D)),
        _full((D, 256)),
        _full((1, 256)),
        _full((256, 64)),
        _full((1, 64)),
    ],
    out_specs=pl.BlockSpec((R, 64), lambda i: (i, 0)),
    out_shape=jax.ShapeDtypeStruct((N, 64), jnp.float32),
)


@jax.jit
def kernel(x, edge_index, W1, b1, W2, b2, LW1, Lb1, LW2, Lb2):
    edge_flat = edge_index.reshape(2 * E)
    zagg = jnp.zeros((RPT, D), jnp.float32)
    zdeg = jnp.zeros((RPT, DEGW), jnp.float32)
    ones8 = jnp.ones((B, DEGW), jnp.float32)
    aggp1, degp = _sc_agg_deg(x, edge_flat, zagg, zdeg, ones8)
    degv = degp.reshape(NC, NP * DEGW // 128, 128)
    h1 = _tc1(aggp1, degv, W1, b1.reshape(1, D))
    aggp2, = _sc_agg(h1, edge_flat, zagg)
    out = _tc2(aggp2, degv, W2, b2.reshape(1, D), LW1, Lb1.reshape(1, 256),
               LW2, Lb2.reshape(1, 64))
    return out


# speedup vs baseline: 1.2028x; 1.0321x over previous
import functools

import jax
import jax.numpy as jnp
from jax import lax
from jax.experimental import pallas as pl
from jax.experimental.pallas import tpu as pltpu
from jax.experimental.pallas import tpu_sc as plsc

N = 10000
NP = 10240
E = 320000
D = 128

NC = 2
NS = 16
NW = NC * NS
EPW = E // NW
B = 80
NB = EPW // B
RPT = NP // NS
ZR = 128
DEGW = 16


DZR = 40


def _sc_agg_body(compute_deg, h_hbm, edge_hbm, agg_out, deg_out,
                 agg_sh, deg_sh, src0, dst0, src1, dst1, src2, dst2,
                 rows0, rows1, rows2, ones_v, dzbuf,
                 isem0, isem1, isem2, gsem0, gsem1, gsem2,
                 ssem0, ssem1, ssem2):
    cid = lax.axis_index("c")
    sid = lax.axis_index("s")
    wid = cid * NS + sid
    base = wid * EPW

    def _zb(i, _):
        r = i // (D // 16)
        c = i % (D // 16)
        rows0[r, pl.ds(c * 16, 16)] = jnp.zeros((16,), jnp.float32)
        return 0
    lax.fori_loop(0, B * (D // 16), _zb, 0)
    for j in range(RPT // B):
        pltpu.sync_copy(rows0, agg_sh.at[pl.ds(sid * RPT + j * B, B)])

    if compute_deg:
        def _ob(i, _):
            ones_v[i] = jnp.ones((16,), jnp.float32)
            return 0
        lax.fori_loop(0, B, _ob, 0)

        def _dz(i, _):
            dzbuf[i] = jnp.zeros((16,), jnp.float32)
            return 0
        lax.fori_loop(0, DZR, _dz, 0)
        for j in range(RPT // DZR):
            pltpu.sync_copy(dzbuf, deg_sh.at[pl.ds(sid * RPT + j * DZR, DZR)])

    plsc.subcore_barrier()

    slots = ((src0, dst0, rows0, isem0, gsem0, ssem0),
             (src1, dst1, rows1, isem1, gsem1, ssem1),
             (src2, dst2, rows2, isem2, gsem2, ssem2))

    def _idx_start(j, s):
        pltpu.async_copy(edge_hbm.at[pl.ds(base + j * B, B)], s[0], s[3])
        pltpu.async_copy(edge_hbm.at[pl.ds(E + base + j * B, B)], s[1], s[3])

    def _idx_wait(j, s):
        pltpu.make_async_copy(edge_hbm.at[pl.ds(base + j * B, B)], s[0],
                              s[3]).wait()
        pltpu.make_async_copy(edge_hbm.at[pl.ds(E + base + j * B, B)], s[1],
                              s[3]).wait()

    def _scatter_start(s):
        pltpu.async_copy(s[2], agg_sh.at[s[1]], s[5], add=True)
        if compute_deg:
            pltpu.async_copy(ones_v, deg_sh.at[s[1]], s[5], add=True)

    def _scatter_wait(s):
        pltpu.make_async_copy(s[2], agg_sh.at[s[1]], s[5]).wait()
        if compute_deg:
            pltpu.make_async_copy(ones_v, deg_sh.at[s[1]], s[5]).wait()

    _idx_start(0, slots[0])
    _idx_start(1, slots[1])
    _idx_wait(0, slots[0])
    pltpu.async_copy(h_hbm.at[slots[0][0]], slots[0][2], slots[0][4])

    def _step(k, s_cur, s_nxt, s_old):
        @pl.when(k + 1 < NB)
        def _():
            _idx_wait(k + 1, s_nxt)
            pltpu.async_copy(h_hbm.at[s_nxt[0]], s_nxt[2], s_nxt[4])
        pltpu.make_async_copy(h_hbm.at[s_cur[0]], s_cur[2], s_cur[4]).wait()
        _scatter_start(s_cur)
        @pl.when(k >= 1)
        def _():
            _scatter_wait(s_old)
        @pl.when(k + 2 < NB)
        def _():
            _idx_start(k + 2, s_old)

    def _tri(g, _):
        for m in range(3):
            k = 3 * g + m
            @pl.when(k < NB)
            def _():
                _step(k, slots[m], slots[(m + 1) % 3], slots[(m + 2) % 3])
        return 0
    lax.fori_loop(0, (NB + 2) // 3, _tri, 0)

    _scatter_wait(slots[(NB - 1) % 3])

    plsc.subcore_barrier()

    pltpu.sync_copy(agg_sh.at[pl.ds(sid * RPT, RPT)],
                    agg_out.at[cid, pl.ds(sid * RPT, RPT)])
    if compute_deg:
        pltpu.sync_copy(deg_sh.at[pl.ds(sid * RPT, RPT)],
                        deg_out.at[cid, pl.ds(sid * RPT, RPT)])


def _make_sc_agg(compute_deg):
    out_type = [jax.ShapeDtypeStruct((NC, NP, D), jnp.float32)]
    if compute_deg:
        out_type.append(jax.ShapeDtypeStruct((NC, NP, DEGW), jnp.float32))
    scratch = [
        pltpu.VMEM_SHARED((NP, D), jnp.float32),
        pltpu.VMEM_SHARED((NP, DEGW), jnp.float32),
        pltpu.VMEM((B,), jnp.int32),
        pltpu.VMEM((B,), jnp.int32),
        pltpu.VMEM((B,), jnp.int32),
        pltpu.VMEM((B,), jnp.int32),
        pltpu.VMEM((B,), jnp.int32),
        pltpu.VMEM((B,), jnp.int32),
        pltpu.VMEM((B, D), jnp.float32),
        pltpu.VMEM((B, D), jnp.float32),
        pltpu.VMEM((B, D), jnp.float32),
        pltpu.VMEM((B, DEGW), jnp.float32),
        pltpu.VMEM((DZR, DEGW), jnp.float32),
        pltpu.SemaphoreType.DMA,
        pltpu.SemaphoreType.DMA,
        pltpu.SemaphoreType.DMA,
        pltpu.SemaphoreType.DMA,
        pltpu.SemaphoreType.DMA,
        pltpu.SemaphoreType.DMA,
        pltpu.SemaphoreType.DMA,
        pltpu.SemaphoreType.DMA,
        pltpu.SemaphoreType.DMA,
    ]
    mesh = plsc.VectorSubcoreMesh(core_axis_name="c", subcore_axis_name="s")
    if compute_deg:
        def body(h, e, agg_o, deg_o, *s):
            _sc_agg_body(True, h, e, agg_o, deg_o, *s)
    else:
        def body(h, e, agg_o, *s):
            _sc_agg_body(False, h, e, agg_o, None, *s)
    return pl.kernel(body, out_type=out_type, mesh=mesh,
                     scratch_types=scratch,
                     compiler_params=pltpu.CompilerParams(
                         use_tc_tiling_on_sc=False))


_sc_agg_deg = _make_sc_agg(True)
_sc_agg = _make_sc_agg(False)


R = 2048


RB = R // 8


def _norm_agg(aggp, degp):
    agg = aggp[0] + aggp[1]
    d = degp[0] + degp[1]
    cc = lax.broadcasted_iota(jnp.int32, (128, 8), 0)
    kk = lax.broadcasted_iota(jnp.int32, (128, 8), 1)
    sel = (cc == kk * DEGW).astype(jnp.float32)
    dv = jnp.dot(d, sel, preferred_element_type=jnp.float32)
    dinv = 1.0 / jnp.maximum(dv, 1.0)
    a3 = agg.reshape(RB, 8, 128) * dinv[:, :, None]
    return a3.reshape(RB * 8, 128)


def _tc1_body(aggp, degp, w, b, out):
    h = jnp.dot(_norm_agg(aggp, degp), w[...],
                preferred_element_type=jnp.float32)
    out[...] = jnp.maximum(h + b[...], 0.0)


def _tc2_body(aggp, degp, w, b, lw1, lb1, lw2, lb2, out):
    h = jnp.dot(_norm_agg(aggp, degp), w[...],
                preferred_element_type=jnp.float32)
    h = jnp.maximum(h + b[...], 0.0)
    t = jnp.dot(h, lw1[...], preferred_element_type=jnp.float32) + lb1[...]
    out[...] = jnp.dot(t, lw2[...], preferred_element_type=jnp.float32) + lb2[...]


def _full(shape):
    return pl.BlockSpec(shape, lambda i: (0,) * len(shape))


_tc1 = pl.pallas_call(
    _tc1_body,
    grid=(NP // R,),
    in_specs=[
        pl.BlockSpec((NC, R, D), lambda i: (0, i, 0)),
        pl.BlockSpec((NC, RB, 128), lambda i: (0, i, 0)),
        _full((D, D)),
        _full((1, D)),
    ],
    out_specs=pl.BlockSpec((R, D), lambda i: (i, 0)),
    out_shape=jax.ShapeDtypeStruct((NP, D), jnp.float32),
)

_tc2 = pl.pallas_call(
    _tc2_body,
    grid=(NP // R,),
    in_specs=[
        pl.BlockSpec((NC, R, D), lambda i: (0, i, 0)),
        pl.BlockSpec((NC, RB, 128), lambda i: (0, i, 0)),
        _full((D, D)),
        _full((1, D)),
        _full((D, 256)),
        _full((1, 256)),
        _full((256, 64)),
        _full((1, 64)),
    ],
    out_specs=pl.BlockSpec((R, 64), lambda i: (i, 0)),
    out_shape=jax.ShapeDtypeStruct((N, 64), jnp.float32),
)


@jax.jit
def kernel(x, edge_index, W1, b1, W2, b2, LW1, Lb1, LW2, Lb2):
    edge_flat = edge_index.reshape(2 * E)
    aggp1, degp = _sc_agg_deg(x, edge_flat)
    degv = degp.reshape(NC, NP * DEGW // 128, 128)
    h1 = _tc1(aggp1, degv, W1, b1.reshape(1, D))
    aggp2, = _sc_agg(h1, edge_flat)
    out = _tc2(aggp2, degv, W2, b2.reshape(1, D), LW1, Lb1.reshape(1, 256),
               LW2, Lb2.reshape(1, 64))
    return out


# speedup vs baseline: 1.2117x; 1.0075x over previous
import functools

import jax
import jax.numpy as jnp
from jax import lax
from jax.experimental import pallas as pl
from jax.experimental.pallas import tpu as pltpu
from jax.experimental.pallas import tpu_sc as plsc

N = 10000
NP = 10240
E = 320000
D = 128

NC = 2
NS = 16
NW = NC * NS
EPW = E // NW
B = 80
NB = EPW // B
RPT = NP // NS
ZR = 128
DEGW = 16


DZR = 40


def _sc_agg_body(compute_deg, h_hbm, edge_hbm, agg_out, deg_out,
                 agg_sh, deg_sh, src0, dst0, src1, dst1, src2, dst2,
                 rows0, rows1, rows2, ones_v, dzbuf,
                 isem0, isem1, isem2, gsem0, gsem1, gsem2,
                 ssem0, ssem1, ssem2):
    cid = lax.axis_index("c")
    sid = lax.axis_index("s")
    wid = cid * NS + sid
    base = wid * EPW

    def _zb(i, _):
        r = i // (D // 16)
        c = i % (D // 16)
        rows0[r, pl.ds(c * 16, 16)] = jnp.zeros((16,), jnp.float32)
        return 0
    lax.fori_loop(0, B * (D // 16), _zb, 0)
    for j in range(RPT // B):
        pltpu.sync_copy(rows0, agg_sh.at[pl.ds(sid * RPT + j * B, B)])

    if compute_deg:
        def _ob(i, _):
            ones_v[i] = jnp.ones((16,), jnp.float32)
            return 0
        lax.fori_loop(0, B, _ob, 0)

        def _dz(i, _):
            dzbuf[i] = jnp.zeros((16,), jnp.float32)
            return 0
        lax.fori_loop(0, DZR, _dz, 0)
        for j in range(RPT // DZR):
            pltpu.sync_copy(dzbuf, deg_sh.at[pl.ds(sid * RPT + j * DZR, DZR)])

    plsc.subcore_barrier()

    slots = ((src0, dst0, rows0, isem0, gsem0, ssem0),
             (src1, dst1, rows1, isem1, gsem1, ssem1),
             (src2, dst2, rows2, isem2, gsem2, ssem2))

    def _idx_start(j, s):
        pltpu.async_copy(edge_hbm.at[pl.ds(base + j * B, B)], s[0], s[3])
        pltpu.async_copy(edge_hbm.at[pl.ds(E + base + j * B, B)], s[1], s[3])

    def _idx_wait(j, s):
        pltpu.make_async_copy(edge_hbm.at[pl.ds(base + j * B, B)], s[0],
                              s[3]).wait()
        pltpu.make_async_copy(edge_hbm.at[pl.ds(E + base + j * B, B)], s[1],
                              s[3]).wait()

    def _scatter_start(s):
        pltpu.async_copy(s[2], agg_sh.at[s[1]], s[5], add=True)
        if compute_deg:
            pltpu.async_copy(ones_v, deg_sh.at[s[1]], s[5], add=True)

    def _scatter_wait(s):
        pltpu.make_async_copy(s[2], agg_sh.at[s[1]], s[5]).wait()
        if compute_deg:
            pltpu.make_async_copy(ones_v, deg_sh.at[s[1]], s[5]).wait()

    _idx_start(0, slots[0])
    _idx_start(1, slots[1])
    _idx_wait(0, slots[0])
    pltpu.async_copy(h_hbm.at[slots[0][0]], slots[0][2], slots[0][4])

    def _step(k, s_cur, s_nxt, s_old):
        @pl.when(k + 1 < NB)
        def _():
            _idx_wait(k + 1, s_nxt)
            pltpu.async_copy(h_hbm.at[s_nxt[0]], s_nxt[2], s_nxt[4])
        pltpu.make_async_copy(h_hbm.at[s_cur[0]], s_cur[2], s_cur[4]).wait()
        _scatter_start(s_cur)
        @pl.when(k >= 1)
        def _():
            _scatter_wait(s_old)
        @pl.when(k + 2 < NB)
        def _():
            _idx_start(k + 2, s_old)

    def _tri(g, _):
        for m in range(3):
            k = 3 * g + m
            @pl.when(k < NB)
            def _():
                _step(k, slots[m], slots[(m + 1) % 3], slots[(m + 2) % 3])
        return 0
    lax.fori_loop(0, (NB + 2) // 3, _tri, 0)

    _scatter_wait(slots[(NB - 1) % 3])

    plsc.subcore_barrier()

    pltpu.sync_copy(agg_sh.at[pl.ds(sid * RPT, RPT)],
                    agg_out.at[cid, pl.ds(sid * RPT, RPT)])
    if compute_deg:
        pltpu.sync_copy(deg_sh.at[pl.ds(sid * RPT, RPT)],
                        deg_out.at[cid, pl.ds(sid * RPT, RPT)])


def _make_sc_agg(compute_deg):
    out_type = [jax.ShapeDtypeStruct((NC, NP, D), jnp.float32)]
    if compute_deg:
        out_type.append(jax.ShapeDtypeStruct((NC, NP, DEGW), jnp.float32))
    scratch = [
        pltpu.VMEM_SHARED((NP, D), jnp.float32),
        pltpu.VMEM_SHARED((NP, DEGW), jnp.float32),
        pltpu.VMEM((B,), jnp.int32),
        pltpu.VMEM((B,), jnp.int32),
        pltpu.VMEM((B,), jnp.int32),
        pltpu.VMEM((B,), jnp.int32),
        pltpu.VMEM((B,), jnp.int32),
        pltpu.VMEM((B,), jnp.int32),
        pltpu.VMEM((B, D), jnp.float32),
        pltpu.VMEM((B, D), jnp.float32),
        pltpu.VMEM((B, D), jnp.float32),
        pltpu.VMEM((B, DEGW), jnp.float32),
        pltpu.VMEM((DZR, DEGW), jnp.float32),
        pltpu.SemaphoreType.DMA,
        pltpu.SemaphoreType.DMA,
        pltpu.SemaphoreType.DMA,
        pltpu.SemaphoreType.DMA,
        pltpu.SemaphoreType.DMA,
        pltpu.SemaphoreType.DMA,
        pltpu.SemaphoreType.DMA,
        pltpu.SemaphoreType.DMA,
        pltpu.SemaphoreType.DMA,
    ]
    mesh = plsc.VectorSubcoreMesh(core_axis_name="c", subcore_axis_name="s")
    if compute_deg:
        def body(h, e, agg_o, deg_o, *s):
            _sc_agg_body(True, h, e, agg_o, deg_o, *s)
    else:
        def body(h, e, agg_o, *s):
            _sc_agg_body(False, h, e, agg_o, None, *s)
    return pl.kernel(body, out_type=out_type, mesh=mesh,
                     scratch_types=scratch,
                     compiler_params=pltpu.CompilerParams(
                         use_tc_tiling_on_sc=False))


_sc_agg_deg = _make_sc_agg(True)
_sc_agg = _make_sc_agg(False)


R = 2560


RB = R // 8


def _norm_agg(aggp, degp):
    agg = aggp[0] + aggp[1]
    d = degp[0] + degp[1]
    cc = lax.broadcasted_iota(jnp.int32, (128, 8), 0)
    kk = lax.broadcasted_iota(jnp.int32, (128, 8), 1)
    sel = (cc == kk * DEGW).astype(jnp.float32)
    dv = jnp.dot(d, sel, preferred_element_type=jnp.float32)
    dinv = 1.0 / jnp.maximum(dv, 1.0)
    a3 = agg.reshape(RB, 8, 128) * dinv[:, :, None]
    return a3.reshape(RB * 8, 128)


def _tc1_body(aggp, degp, w, b, out):
    h = jnp.dot(_norm_agg(aggp, degp), w[...],
                preferred_element_type=jnp.float32)
    out[...] = jnp.maximum(h + b[...], 0.0)


def _tc2_body(aggp, degp, w, b, lw1, lb1, lw2, lb2, out):
    h = jnp.dot(_norm_agg(aggp, degp), w[...],
                preferred_element_type=jnp.float32)
    h = jnp.maximum(h + b[...], 0.0)
    t = jnp.dot(h, lw1[...], preferred_element_type=jnp.float32) + lb1[...]
    out[...] = jnp.dot(t, lw2[...], preferred_element_type=jnp.float32) + lb2[...]


def _full(shape):
    return pl.BlockSpec(shape, lambda i: (0,) * len(shape))


_tc1 = pl.pallas_call(
    _tc1_body,
    grid=(NP // R,),
    in_specs=[
        pl.BlockSpec((NC, R, D), lambda i: (0, i, 0)),
        pl.BlockSpec((NC, RB, 128), lambda i: (0, i, 0)),
        _full((D, D)),
        _full((1, D)),
    ],
    out_specs=pl.BlockSpec((R, D), lambda i: (i, 0)),
    out_shape=jax.ShapeDtypeStruct((NP, D), jnp.float32),
)

_tc2 = pl.pallas_call(
    _tc2_body,
    grid=(NP // R,),
    in_specs=[
        pl.BlockSpec((NC, R, D), lambda i: (0, i, 0)),
        pl.BlockSpec((NC, RB, 128), lambda i: (0, i, 0)),
        _full((D, D)),
        _full((1, D)),
        _full((D, 256)),
        _full((1, 256)),
        _full((256, 64)),
        _full((1, 64)),
    ],
    out_specs=pl.BlockSpec((R, 64), lambda i: (i, 0)),
    out_shape=jax.ShapeDtypeStruct((N, 64), jnp.float32),
)


@jax.jit
def kernel(x, edge_index, W1, b1, W2, b2, LW1, Lb1, LW2, Lb2):
    edge_flat = edge_index.reshape(2 * E)
    aggp1, degp = _sc_agg_deg(x, edge_flat)
    degv = degp.reshape(NC, NP * DEGW // 128, 128)
    h1 = _tc1(aggp1, degv, W1, b1.reshape(1, D))
    aggp2, = _sc_agg(h1, edge_flat)
    out = _tc2(aggp2, degv, W2, b2.reshape(1, D), LW1, Lb1.reshape(1, 256),
               LW2, Lb2.reshape(1, 64))
    return out


# speedup vs baseline: 1.2187x; 1.0058x over previous
import functools

import jax
import jax.numpy as jnp
from jax import lax
from jax.experimental import pallas as pl
from jax.experimental.pallas import tpu as pltpu
from jax.experimental.pallas import tpu_sc as plsc

N = 10000
NP = 10240
E = 320000
D = 128

NC = 2
NS = 16
NW = NC * NS
EPW = E // NW
B = 80
NB = EPW // B
RPT = NP // NS
ZR = 128
DEGW = 16


DZR = 40


def _sc_agg_body(compute_deg, h_hbm, edge_hbm, agg_out, deg_out,
                 agg_sh, deg_sh, src0, dst0, src1, dst1, src2, dst2,
                 rows0, rows1, rows2, ones_v, dzbuf,
                 isem0, isem1, isem2, gsem0, gsem1, gsem2,
                 ssem0, ssem1, ssem2):
    cid = lax.axis_index("c")
    sid = lax.axis_index("s")
    wid = cid * NS + sid
    base = wid * EPW

    def _zb(i, _):
        r = i // (D // 16)
        c = i % (D // 16)
        rows0[r, pl.ds(c * 16, 16)] = jnp.zeros((16,), jnp.float32)
        return 0
    lax.fori_loop(0, B * (D // 16), _zb, 0)
    for j in range(RPT // B):
        pltpu.sync_copy(rows0, agg_sh.at[pl.ds(sid * RPT + j * B, B)])

    if compute_deg:
        def _ob(i, _):
            ones_v[i] = jnp.ones((16,), jnp.float32)
            return 0
        lax.fori_loop(0, B, _ob, 0)

        def _dz(i, _):
            dzbuf[i] = jnp.zeros((16,), jnp.float32)
            return 0
        lax.fori_loop(0, DZR, _dz, 0)
        for j in range(RPT // DZR):
            pltpu.sync_copy(dzbuf, deg_sh.at[pl.ds(sid * RPT + j * DZR, DZR)])

    plsc.subcore_barrier()

    slots = ((src0, dst0, rows0, isem0, gsem0, ssem0),
             (src1, dst1, rows1, isem1, gsem1, ssem1),
             (src2, dst2, rows2, isem2, gsem2, ssem2))

    def _idx_start(j, s):
        pltpu.async_copy(edge_hbm.at[pl.ds(base + j * B, B)], s[0], s[3])
        pltpu.async_copy(edge_hbm.at[pl.ds(E + base + j * B, B)], s[1], s[3])

    def _idx_wait(j, s):
        pltpu.make_async_copy(edge_hbm.at[pl.ds(base + j * B, B)], s[0],
                              s[3]).wait()
        pltpu.make_async_copy(edge_hbm.at[pl.ds(E + base + j * B, B)], s[1],
                              s[3]).wait()

    def _scatter_start(s):
        pltpu.async_copy(s[2], agg_sh.at[s[1]], s[5], add=True)
        if compute_deg:
            pltpu.async_copy(ones_v, deg_sh.at[s[1]], s[5], add=True)

    def _scatter_wait(s):
        pltpu.make_async_copy(s[2], agg_sh.at[s[1]], s[5]).wait()
        if compute_deg:
            pltpu.make_async_copy(ones_v, deg_sh.at[s[1]], s[5]).wait()

    _idx_start(0, slots[0])
    _idx_start(1, slots[1])
    _idx_wait(0, slots[0])
    pltpu.async_copy(h_hbm.at[slots[0][0]], slots[0][2], slots[0][4])

    def _step(k, s_cur, s_nxt, s_old):
        @pl.when(k + 1 < NB)
        def _():
            _idx_wait(k + 1, s_nxt)
            pltpu.async_copy(h_hbm.at[s_nxt[0]], s_nxt[2], s_nxt[4])
        pltpu.make_async_copy(h_hbm.at[s_cur[0]], s_cur[2], s_cur[4]).wait()
        _scatter_start(s_cur)
        @pl.when(k >= 1)
        def _():
            _scatter_wait(s_old)
        @pl.when(k + 2 < NB)
        def _():
            _idx_start(k + 2, s_old)

    def _tri(g, _):
        for m in range(3):
            k = 3 * g + m
            @pl.when(k < NB)
            def _():
                _step(k, slots[m], slots[(m + 1) % 3], slots[(m + 2) % 3])
        return 0
    lax.fori_loop(0, (NB + 2) // 3, _tri, 0)

    _scatter_wait(slots[(NB - 1) % 3])

    plsc.subcore_barrier()

    pltpu.sync_copy(agg_sh.at[pl.ds(sid * RPT, RPT)],
                    agg_out.at[cid, pl.ds(sid * RPT, RPT)])
    if compute_deg:
        pltpu.sync_copy(deg_sh.at[pl.ds(sid * RPT, RPT)],
                        deg_out.at[cid, pl.ds(sid * RPT, RPT)])


def _make_sc_agg(compute_deg):
    out_type = [jax.ShapeDtypeStruct((NC, NP, D), jnp.float32)]
    if compute_deg:
        out_type.append(jax.ShapeDtypeStruct((NC, NP, DEGW), jnp.float32))
    scratch = [
        pltpu.VMEM_SHARED((NP, D), jnp.float32),
        pltpu.VMEM_SHARED((NP, DEGW), jnp.float32),
        pltpu.VMEM((B,), jnp.int32),
        pltpu.VMEM((B,), jnp.int32),
        pltpu.VMEM((B,), jnp.int32),
        pltpu.VMEM((B,), jnp.int32),
        pltpu.VMEM((B,), jnp.int32),
        pltpu.VMEM((B,), jnp.int32),
        pltpu.VMEM((B, D), jnp.float32),
        pltpu.VMEM((B, D), jnp.float32),
        pltpu.VMEM((B, D), jnp.float32),
        pltpu.VMEM((B, DEGW), jnp.float32),
        pltpu.VMEM((DZR, DEGW), jnp.float32),
        pltpu.SemaphoreType.DMA,
        pltpu.SemaphoreType.DMA,
        pltpu.SemaphoreType.DMA,
        pltpu.SemaphoreType.DMA,
        pltpu.SemaphoreType.DMA,
        pltpu.SemaphoreType.DMA,
        pltpu.SemaphoreType.DMA,
        pltpu.SemaphoreType.DMA,
        pltpu.SemaphoreType.DMA,
    ]
    mesh = plsc.VectorSubcoreMesh(core_axis_name="c", subcore_axis_name="s")
    if compute_deg:
        def body(h, e, agg_o, deg_o, *s):
            _sc_agg_body(True, h, e, agg_o, deg_o, *s)
    else:
        def body(h, e, agg_o, *s):
            _sc_agg_body(False, h, e, agg_o, None, *s)
    return pl.kernel(body, out_type=out_type, mesh=mesh,
                     scratch_types=scratch,
                     compiler_params=pltpu.CompilerParams(
                         use_tc_tiling_on_sc=False))


_sc_agg_deg = _make_sc_agg(True)
_sc_agg = _make_sc_agg(False)


R = 5120


RB = R // 8


def _norm_agg(aggp, degp):
    agg = aggp[0] + aggp[1]
    d = degp[0] + degp[1]
    cc = lax.broadcasted_iota(jnp.int32, (128, 8), 0)
    kk = lax.broadcasted_iota(jnp.int32, (128, 8), 1)
    sel = (cc == kk * DEGW).astype(jnp.float32)
    dv = jnp.dot(d, sel, preferred_element_type=jnp.float32)
    dinv = 1.0 / jnp.maximum(dv, 1.0)
    a3 = agg.reshape(RB, 8, 128) * dinv[:, :, None]
    return a3.reshape(RB * 8, 128)


def _tc1_body(aggp, degp, w, b, out):
    h = jnp.dot(_norm_agg(aggp, degp), w[...],
                preferred_element_type=jnp.float32)
    out[...] = jnp.maximum(h + b[...], 0.0)


def _tc2_body(aggp, degp, w, b, lw1, lb1, lw2, lb2, out):
    h = jnp.dot(_norm_agg(aggp, degp), w[...],
                preferred_element_type=jnp.float32)
    h = jnp.maximum(h + b[...], 0.0)
    t = jnp.dot(h, lw1[...], preferred_element_type=jnp.float32) + lb1[...]
    out[...] = jnp.dot(t, lw2[...], preferred_element_type=jnp.float32) + lb2[...]


def _full(shape):
    return pl.BlockSpec(shape, lambda i: (0,) * len(shape))


_tc1 = pl.pallas_call(
    _tc1_body,
    grid=(NP // R,),
    in_specs=[
        pl.BlockSpec((NC, R, D), lambda i: (0, i, 0)),
        pl.BlockSpec((NC, RB, 128), lambda i: (0, i, 0)),
        _full((D, D)),
        _full((1, D)),
    ],
    out_specs=pl.BlockSpec((R, D), lambda i: (i, 0)),
    out_shape=jax.ShapeDtypeStruct((NP, D), jnp.float32),
)

_tc2 = pl.pallas_call(
    _tc2_body,
    grid=(NP // R,),
    in_specs=[
        pl.BlockSpec((NC, R, D), lambda i: (0, i, 0)),
        pl.BlockSpec((NC, RB, 128), lambda i: (0, i, 0)),
        _full((D, D)),
        _full((1, D)),
        _full((D, 256)),
        _full((1, 256)),
        _full((256, 64)),
        _full((1, 64)),
    ],
    out_specs=pl.BlockSpec((R, 64), lambda i: (i, 0)),
    out_shape=jax.ShapeDtypeStruct((N, 64), jnp.float32),
)


@jax.jit
def kernel(x, edge_index, W1, b1, W2, b2, LW1, Lb1, LW2, Lb2):
    edge_flat = edge_index.reshape(2 * E)
    aggp1, degp = _sc_agg_deg(x, edge_flat)
    degv = degp.reshape(NC, NP * DEGW // 128, 128)
    h1 = _tc1(aggp1, degv, W1, b1.reshape(1, D))
    aggp2, = _sc_agg(h1, edge_flat)
    out = _tc2(aggp2, degv, W2, b2.reshape(1, D), LW1, Lb1.reshape(1, 256),
               LW2, Lb2.reshape(1, 64))
    return out
